# Initial kernel scaffold; baseline (speedup 1.0000x reference)
import jax
import jax.numpy as jnp
from jax.experimental import pallas as pl


def kernel(op_idx, col_idx, sec_idx, flag, op_emb, col_emb):
    raise NotImplementedError("write your pallas kernel here")


# speedup vs baseline: 1.6403x; 1.6403x over previous
import functools

import jax
import jax.numpy as jnp
from jax import lax
from jax.experimental import pallas as pl
from jax.experimental.pallas import tpu as pltpu
from jax.experimental.pallas import tpu_sc as plsc

B = 16384
NC = 2
NS = 16
NW = NC * NS
BPW = B // NW
NCHUNK = BPW // 128
OUT_D = 20

_mesh = plsc.VectorSubcoreMesh(
    core_axis_name="c", subcore_axis_name="s", num_cores=NC, num_subcores=NS
)


@functools.partial(
    pl.kernel,
    out_type=jax.ShapeDtypeStruct((B, OUT_D), jnp.float32),
    mesh=_mesh,
    compiler_params=pltpu.CompilerParams(
        needs_layout_passes=False, use_tc_tiling_on_sc=False),
    scratch_types=[
        pltpu.VMEM((BPW,), jnp.int32),
        pltpu.VMEM((NCHUNK, 128), jnp.int32),
        pltpu.VMEM((NCHUNK, 128), jnp.int32),
        pltpu.VMEM((BPW,), jnp.float32),
        pltpu.VMEM((6, 3), jnp.float32),
        pltpu.VMEM((BPW, 8), jnp.float32),
        pltpu.VMEM((BPW, 8), jnp.float32),
        pltpu.VMEM((BPW, OUT_D), jnp.float32),
        pltpu.SemaphoreType.DMA,
    ],
)
def _encode(op_idx_h, col_idx_h, sec_idx_h, flag_h, op_emb_h, col_emb_h,
            out_h, opi_v, coli_v, seci_v, flag_v, opt_v, colr_v, secr_v,
            out_v, sem):
    wid = lax.axis_index("s") * NC + lax.axis_index("c")
    base = wid * BPW

    for c in range(NCHUNK):
        pltpu.sync_copy(col_idx_h.at[pl.ds(base + c * 128, 128)], coli_v.at[c])
        pltpu.sync_copy(sec_idx_h.at[pl.ds(base + c * 128, 128)], seci_v.at[c])
    copies = []
    for c in range(NCHUNK):
        copies.append(pltpu.async_copy(
            col_emb_h.at[coli_v.at[c]], colr_v.at[pl.ds(c * 128, 128)], sem))
        copies.append(pltpu.async_copy(
            col_emb_h.at[seci_v.at[c]], secr_v.at[pl.ds(c * 128, 128)], sem))

    pltpu.sync_copy(op_idx_h.at[pl.ds(base, BPW)], opi_v)
    pltpu.sync_copy(flag_h.at[pl.ds(base, BPW)], flag_v)
    pltpu.sync_copy(op_emb_h, opt_v)
    for cp in copies:
        cp.wait()

    def step(i, carry):
        rows = i * 16 + lax.iota(jnp.int32, 16)
        o = plsc.load_gather(opi_v, [rows])
        for j in range(3):
            cj = jnp.full((16,), j, jnp.int32)
            v = plsc.load_gather(opt_v, [o, cj])
            plsc.store_scatter(out_v, [rows, cj], v)
        for j in range(8):
            cj = jnp.full((16,), j, jnp.int32)
            v = plsc.load_gather(colr_v, [rows, cj])
            plsc.store_scatter(out_v, [rows, jnp.full((16,), 3 + j, jnp.int32)], v)
            w = plsc.load_gather(secr_v, [rows, cj])
            plsc.store_scatter(out_v, [rows, jnp.full((16,), 11 + j, jnp.int32)], w)
        f = plsc.load_gather(flag_v, [rows])
        plsc.store_scatter(out_v, [rows, jnp.full((16,), 19, jnp.int32)], f)
        return carry

    lax.fori_loop(0, BPW // 16, step, 0)

    pltpu.sync_copy(out_v, out_h.at[pl.ds(base, BPW)])


def kernel(op_idx, col_idx, sec_idx, flag, op_emb, col_emb):
    return _encode(
        op_idx.astype(jnp.int32),
        col_idx.astype(jnp.int32),
        sec_idx.astype(jnp.int32),
        flag.reshape(B).astype(jnp.float32),
        op_emb,
        col_emb,
    )


# speedup vs baseline: 1.6411x; 1.0005x over previous
import functools

import jax
import jax.numpy as jnp
from jax import lax
from jax.experimental import pallas as pl
from jax.experimental.pallas import tpu as pltpu
from jax.experimental.pallas import tpu_sc as plsc

B = 16384
NC = 2
NS = 16
NW = NC * NS
BPW = B // NW
NCHUNK = BPW // 128
OUT_D = 20

_mesh = plsc.VectorSubcoreMesh(
    core_axis_name="c", subcore_axis_name="s", num_cores=NC, num_subcores=NS
)


@functools.partial(
    pl.kernel,
    out_type=jax.ShapeDtypeStruct((B, OUT_D), jnp.float32),
    mesh=_mesh,
    compiler_params=pltpu.CompilerParams(
        needs_layout_passes=False, use_tc_tiling_on_sc=False,
        skip_device_barrier=True, disable_bounds_checks=True,
        disable_semaphore_checks=True),
    scratch_types=[
        pltpu.VMEM((BPW,), jnp.int32),
        pltpu.VMEM((NCHUNK, 128), jnp.int32),
        pltpu.VMEM((NCHUNK, 128), jnp.int32),
        pltpu.VMEM((BPW,), jnp.float32),
        pltpu.VMEM((6, 3), jnp.float32),
        pltpu.VMEM((BPW, 8), jnp.float32),
        pltpu.VMEM((BPW, 8), jnp.float32),
        pltpu.VMEM((BPW, OUT_D), jnp.float32),
        pltpu.SemaphoreType.DMA,
    ],
)
def _encode(op_idx_h, col_idx_h, sec_idx_h, flag_h, op_emb_h, col_emb_h,
            out_h, opi_v, coli_v, seci_v, flag_v, opt_v, colr_v, secr_v,
            out_v, sem):
    wid = lax.axis_index("s") * NC + lax.axis_index("c")
    base = wid * BPW

    for c in range(NCHUNK):
        pltpu.sync_copy(col_idx_h.at[pl.ds(base + c * 128, 128)], coli_v.at[c])
        pltpu.sync_copy(sec_idx_h.at[pl.ds(base + c * 128, 128)], seci_v.at[c])
    copies = []
    for c in range(NCHUNK):
        copies.append(pltpu.async_copy(
            col_emb_h.at[coli_v.at[c]], colr_v.at[pl.ds(c * 128, 128)], sem))
        copies.append(pltpu.async_copy(
            col_emb_h.at[seci_v.at[c]], secr_v.at[pl.ds(c * 128, 128)], sem))

    pltpu.sync_copy(op_idx_h.at[pl.ds(base, BPW)], opi_v)
    pltpu.sync_copy(flag_h.at[pl.ds(base, BPW)], flag_v)
    pltpu.sync_copy(op_emb_h, opt_v)
    for cp in copies:
        cp.wait()

    def step(i, carry):
        rows = i * 16 + lax.iota(jnp.int32, 16)
        o = plsc.load_gather(opi_v, [rows])
        for j in range(3):
            cj = jnp.full((16,), j, jnp.int32)
            v = plsc.load_gather(opt_v, [o, cj])
            plsc.store_scatter(out_v, [rows, cj], v)
        for j in range(8):
            cj = jnp.full((16,), j, jnp.int32)
            v = plsc.load_gather(colr_v, [rows, cj])
            plsc.store_scatter(out_v, [rows, jnp.full((16,), 3 + j, jnp.int32)], v)
            w = plsc.load_gather(secr_v, [rows, cj])
            plsc.store_scatter(out_v, [rows, jnp.full((16,), 11 + j, jnp.int32)], w)
        f = plsc.load_gather(flag_v, [rows])
        plsc.store_scatter(out_v, [rows, jnp.full((16,), 19, jnp.int32)], f)
        return carry

    lax.fori_loop(0, BPW // 16, step, 0)

    pltpu.sync_copy(out_v, out_h.at[pl.ds(base, BPW)])


def kernel(op_idx, col_idx, sec_idx, flag, op_emb, col_emb):
    return _encode(
        op_idx.astype(jnp.int32),
        col_idx.astype(jnp.int32),
        sec_idx.astype(jnp.int32),
        flag.reshape(B).astype(jnp.float32),
        op_emb,
        col_emb,
    )


# speedup vs baseline: 3.0433x; 1.8545x over previous
import functools

import jax
import jax.numpy as jnp
from jax import lax
from jax.experimental import pallas as pl
from jax.experimental.pallas import tpu as pltpu
from jax.experimental.pallas import tpu_sc as plsc

B = 16384
NC = 2
NS = 16
NW = NC * NS
BPW = B // NW
V = 100000
NBLK = 49
NBLK_LAST = 46
CH = NBLK * 128
TAIL = V - 15 * CH - NBLK_LAST * 128
OUT_D = 20

_mesh = plsc.VectorSubcoreMesh(
    core_axis_name="c", subcore_axis_name="s", num_cores=NC, num_subcores=NS
)


@functools.partial(
    pl.kernel,
    out_type=jax.ShapeDtypeStruct((OUT_D, B), jnp.float32),
    mesh=_mesh,
    compiler_params=pltpu.CompilerParams(
        needs_layout_passes=False, use_tc_tiling_on_sc=True,
        skip_device_barrier=True, disable_bounds_checks=True,
        disable_semaphore_checks=True),
    scratch_types=[
        pltpu.VMEM_SHARED((8 * V,), jnp.float32),
        pltpu.VMEM((8, 128), jnp.float32),
        pltpu.VMEM((8, 128), jnp.float32),
        pltpu.VMEM((8, TAIL), jnp.float32),
        pltpu.VMEM((BPW,), jnp.int32),
        pltpu.VMEM((BPW,), jnp.int32),
        pltpu.VMEM((BPW,), jnp.int32),
        pltpu.VMEM((8, BPW), jnp.int32),
        pltpu.VMEM((8, BPW), jnp.int32),
        pltpu.VMEM((6, 3), jnp.float32),
        pltpu.VMEM((OUT_D, BPW), jnp.float32),
        pltpu.SemaphoreType.DMA,
        pltpu.SemaphoreType.DMA,
        pltpu.SemaphoreType.DMA,
    ],
)
def _encode(op_idx_h, col_idx_h, sec_idx_h, flag_h, op_emb_h, col_t_h,
            tail_h, out_h, shared, buf0_v, buf1_v, tail_v, opi_v, coli_v,
            seci_v, cola_v, seca_v, opt_v, out_v, gsem, hsem, ssem):
    cid = lax.axis_index("c")
    sid = lax.axis_index("s")
    wid = sid * NC + cid
    base = wid * BPW

    def stage_table(nblk):
        bufs = (buf0_v, buf1_v)
        col0 = sid * CH

        def fetch(b):
            return pltpu.async_copy(
                col_t_h.at[:, pl.ds(col0 + b * 128, 128)], bufs[b % 2], hsem)

        def push(b):
            cps = []
            for j in range(8):
                cps.append(pltpu.async_copy(
                    bufs[b % 2].at[j],
                    shared.at[pl.ds(j * V + col0 + b * 128, 128)], ssem))
            return cps

        pend_fetch = fetch(0)
        pend_push = None
        for b in range(nblk):
            if pend_push is not None:
                for cp in pend_push:
                    cp.wait()
            nxt = fetch(b + 1) if b + 1 < nblk else None
            pend_fetch.wait()
            pend_push = push(b)
            pend_fetch = nxt
        for cp in pend_push:
            cp.wait()

    @pl.when(sid < 15)
    def _():
        stage_table(NBLK)

    @pl.when(sid == 15)
    def _():
        stage_table(NBLK_LAST)
        pltpu.sync_copy(tail_h, tail_v)
        for j in range(8):
            pltpu.sync_copy(
                tail_v.at[j],
                shared.at[pl.ds(j * V + sid * CH + NBLK_LAST * 128, TAIL)])

    pltpu.sync_copy(col_idx_h.at[pl.ds(base, BPW)], coli_v)
    pltpu.sync_copy(sec_idx_h.at[pl.ds(base, BPW)], seci_v)
    pltpu.sync_copy(op_idx_h.at[pl.ds(base, BPW)], opi_v)
    pltpu.sync_copy(op_emb_h, opt_v)
    pltpu.sync_copy(flag_h.at[pl.ds(base, BPW)], out_v.at[19])

    def prep_step(i, carry):
        ci = coli_v[pl.ds(i * 16, 16)]
        si = seci_v[pl.ds(i * 16, 16)]
        for j in range(8):
            cola_v[j, pl.ds(i * 16, 16)] = ci + j * V
            seca_v[j, pl.ds(i * 16, 16)] = si + j * V
        o = opi_v[pl.ds(i * 16, 16)]
        for j in range(3):
            v = plsc.load_gather(opt_v, [o, jnp.full((16,), j, jnp.int32)])
            out_v[j, pl.ds(i * 16, 16)] = v
        return carry

    lax.fori_loop(0, BPW // 16, prep_step, 0)

    plsc.subcore_barrier()

    copies = []
    for j in range(8):
        for c in range(BPW // 128):
            copies.append(pltpu.async_copy(
                shared.at[cola_v.at[j, pl.ds(c * 128, 128)]],
                out_v.at[3 + j, pl.ds(c * 128, 128)], gsem))
            copies.append(pltpu.async_copy(
                shared.at[seca_v.at[j, pl.ds(c * 128, 128)]],
                out_v.at[11 + j, pl.ds(c * 128, 128)], gsem))
    for cp in copies:
        cp.wait()

    pltpu.sync_copy(out_v, out_h.at[:, pl.ds(base, BPW)])


def kernel(op_idx, col_idx, sec_idx, flag, op_emb, col_emb):
    col_t = col_emb.T
    out_t = _encode(
        op_idx.astype(jnp.int32),
        col_idx.astype(jnp.int32),
        sec_idx.astype(jnp.int32),
        flag.reshape(B).astype(jnp.float32),
        op_emb,
        col_t,
        col_t[:, 15 * CH + NBLK_LAST * 128:],
    )
    return out_t.T


# speedup vs baseline: 3.7870x; 1.2444x over previous
import functools

import jax
import jax.numpy as jnp
from jax import lax
from jax.experimental import pallas as pl
from jax.experimental.pallas import tpu as pltpu
from jax.experimental.pallas import tpu_sc as plsc

B = 16384
NC = 2
NS = 16
NW = NC * NS
BPW = B // NW
V = 100000
NBLK = 49
NBLK_LAST = 46
CH = NBLK * 128
TAIL = V - 15 * CH - NBLK_LAST * 128
OUT_D = 20

_mesh = plsc.VectorSubcoreMesh(
    core_axis_name="c", subcore_axis_name="s", num_cores=NC, num_subcores=NS
)


@functools.partial(
    pl.kernel,
    out_type=jax.ShapeDtypeStruct((OUT_D, B), jnp.float32),
    mesh=_mesh,
    compiler_params=pltpu.CompilerParams(
        needs_layout_passes=False, use_tc_tiling_on_sc=True,
        skip_device_barrier=True, disable_bounds_checks=True,
        disable_semaphore_checks=True),
    scratch_types=[
        pltpu.VMEM_SHARED((8 * V,), jnp.float32),
        [pltpu.VMEM((8, 128), jnp.float32)] * 8,
        pltpu.VMEM((8, TAIL), jnp.float32),
        pltpu.VMEM((BPW,), jnp.int32),
        pltpu.VMEM((BPW,), jnp.int32),
        pltpu.VMEM((BPW,), jnp.int32),
        pltpu.VMEM((8, BPW), jnp.int32),
        pltpu.VMEM((8, BPW), jnp.int32),
        pltpu.VMEM((6, 3), jnp.float32),
        pltpu.VMEM((OUT_D, BPW), jnp.float32),
        pltpu.SemaphoreType.DMA,
        pltpu.SemaphoreType.DMA,
        pltpu.SemaphoreType.DMA,
    ],
)
def _encode(op_idx_h, col_idx_h, sec_idx_h, flag_h, op_emb_h, col_t_h,
            tail_h, out_h, shared, bufs, tail_v, opi_v, coli_v,
            seci_v, cola_v, seca_v, opt_v, out_v, gsem, hsem, ssem):
    cid = lax.axis_index("c")
    sid = lax.axis_index("s")
    wid = sid * NC + cid
    base = wid * BPW

    def stage_table(nblk):
        nbuf = len(bufs)
        col0 = sid * CH

        def fetch(b):
            return pltpu.async_copy(
                col_t_h.at[:, pl.ds(col0 + b * 128, 128)], bufs[b % nbuf],
                hsem)

        def push(b):
            cps = []
            for j in range(8):
                cps.append(pltpu.async_copy(
                    bufs[b % nbuf].at[j],
                    shared.at[pl.ds(j * V + col0 + b * 128, 128)], ssem))
            return cps

        window = nbuf - 1
        fetches = [fetch(b) for b in range(min(window, nblk))]
        pushes = {}
        for b in range(nblk):
            fetches[b].wait()
            pushes[b] = push(b)
            f = b + window
            if f < nblk:
                if f - nbuf >= 0:
                    for cp in pushes.pop(f - nbuf):
                        cp.wait()
                fetches.append(fetch(f))
        for cps in pushes.values():
            for cp in cps:
                cp.wait()

    @pl.when(sid < 15)
    def _():
        stage_table(NBLK)

    @pl.when(sid == 15)
    def _():
        stage_table(NBLK_LAST)
        pltpu.sync_copy(tail_h, tail_v)
        for j in range(8):
            pltpu.sync_copy(
                tail_v.at[j],
                shared.at[pl.ds(j * V + sid * CH + NBLK_LAST * 128, TAIL)])

    pltpu.sync_copy(col_idx_h.at[pl.ds(base, BPW)], coli_v)
    pltpu.sync_copy(sec_idx_h.at[pl.ds(base, BPW)], seci_v)
    pltpu.sync_copy(op_idx_h.at[pl.ds(base, BPW)], opi_v)
    pltpu.sync_copy(op_emb_h, opt_v)
    pltpu.sync_copy(flag_h.at[pl.ds(base, BPW)], out_v.at[19])

    def prep_step(i, carry):
        ci = coli_v[pl.ds(i * 16, 16)]
        si = seci_v[pl.ds(i * 16, 16)]
        for j in range(8):
            cola_v[j, pl.ds(i * 16, 16)] = ci + j * V
            seca_v[j, pl.ds(i * 16, 16)] = si + j * V
        o = opi_v[pl.ds(i * 16, 16)]
        for j in range(3):
            v = plsc.load_gather(opt_v, [o, jnp.full((16,), j, jnp.int32)])
            out_v[j, pl.ds(i * 16, 16)] = v
        return carry

    lax.fori_loop(0, BPW // 16, prep_step, 0)

    plsc.subcore_barrier()

    copies = []
    for j in range(8):
        for c in range(BPW // 128):
            copies.append(pltpu.async_copy(
                shared.at[cola_v.at[j, pl.ds(c * 128, 128)]],
                out_v.at[3 + j, pl.ds(c * 128, 128)], gsem))
            copies.append(pltpu.async_copy(
                shared.at[seca_v.at[j, pl.ds(c * 128, 128)]],
                out_v.at[11 + j, pl.ds(c * 128, 128)], gsem))
    for cp in copies:
        cp.wait()

    pltpu.sync_copy(out_v, out_h.at[:, pl.ds(base, BPW)])


def kernel(op_idx, col_idx, sec_idx, flag, op_emb, col_emb):
    col_t = col_emb.T
    out_t = _encode(
        op_idx.astype(jnp.int32),
        col_idx.astype(jnp.int32),
        sec_idx.astype(jnp.int32),
        flag.reshape(B).astype(jnp.float32),
        op_emb,
        col_t,
        col_t[:, 15 * CH + NBLK_LAST * 128:],
    )
    return out_t.T


# speedup vs baseline: 3.9203x; 1.0352x over previous
import functools

import jax
import jax.numpy as jnp
from jax import lax
from jax.experimental import pallas as pl
from jax.experimental.pallas import tpu as pltpu
from jax.experimental.pallas import tpu_sc as plsc

B = 16384
NC = 2
NS = 16
NW = NC * NS
BPW = B // NW
V = 100000
NBLK = 49
NBLK_LAST = 46
CH = NBLK * 128
TAIL = V - 15 * CH - NBLK_LAST * 128
OUT_D = 20

_mesh = plsc.VectorSubcoreMesh(
    core_axis_name="c", subcore_axis_name="s", num_cores=NC, num_subcores=NS
)


@functools.partial(
    pl.kernel,
    out_type=jax.ShapeDtypeStruct((OUT_D, B), jnp.float32),
    mesh=_mesh,
    compiler_params=pltpu.CompilerParams(
        needs_layout_passes=False, use_tc_tiling_on_sc=True,
        skip_device_barrier=True, disable_bounds_checks=True,
        disable_semaphore_checks=True),
    scratch_types=[
        pltpu.VMEM_SHARED((8 * V,), jnp.float32),
        [pltpu.VMEM((8, 128), jnp.float32)] * NBLK,
        pltpu.VMEM((8, TAIL), jnp.float32),
        pltpu.VMEM((BPW,), jnp.int32),
        pltpu.VMEM((BPW,), jnp.int32),
        pltpu.VMEM((BPW,), jnp.int32),
        pltpu.VMEM((8, BPW), jnp.int32),
        pltpu.VMEM((8, BPW), jnp.int32),
        pltpu.VMEM((6, 3), jnp.float32),
        pltpu.VMEM((OUT_D, BPW), jnp.float32),
        pltpu.SemaphoreType.DMA,
        pltpu.SemaphoreType.DMA,
        pltpu.SemaphoreType.DMA,
    ],
)
def _encode(op_idx_h, col_idx_h, sec_idx_h, flag_h, op_emb_h, col_t_h,
            tail_h, out_h, shared, bufs, tail_v, opi_v, coli_v,
            seci_v, cola_v, seca_v, opt_v, out_v, gsem, hsem, ssem):
    cid = lax.axis_index("c")
    sid = lax.axis_index("s")
    wid = sid * NC + cid
    base = wid * BPW
    col0 = sid * CH

    def fetch_all(nblk):
        cps = []
        for b in range(nblk):
            cps.append(pltpu.async_copy(
                col_t_h.at[:, pl.ds(col0 + b * 128, 128)], bufs[b], hsem))
        return cps

    @pl.when(sid < 15)
    def _():
        for cp in fetch_all(NBLK):
            cp.wait()

    @pl.when(sid == 15)
    def _():
        cps = fetch_all(NBLK_LAST)
        pltpu.sync_copy(tail_h, tail_v)
        for cp in cps:
            cp.wait()

    pltpu.sync_copy(col_idx_h.at[pl.ds(base, BPW)], coli_v)
    pltpu.sync_copy(sec_idx_h.at[pl.ds(base, BPW)], seci_v)
    pltpu.sync_copy(op_idx_h.at[pl.ds(base, BPW)], opi_v)
    pltpu.sync_copy(op_emb_h, opt_v)
    pltpu.sync_copy(flag_h.at[pl.ds(base, BPW)], out_v.at[19])

    def prep_step(i, carry):
        ci = coli_v[pl.ds(i * 16, 16)]
        si = seci_v[pl.ds(i * 16, 16)]
        for j in range(8):
            cola_v[j, pl.ds(i * 16, 16)] = ci + j * V
            seca_v[j, pl.ds(i * 16, 16)] = si + j * V
        o = opi_v[pl.ds(i * 16, 16)]
        for j in range(3):
            v = plsc.load_gather(opt_v, [o, jnp.full((16,), j, jnp.int32)])
            out_v[j, pl.ds(i * 16, 16)] = v
        return carry

    lax.fori_loop(0, BPW // 16, prep_step, 0)

    gathers = []
    for j in range(8):
        def push_row(nblk):
            cps = []
            for b in range(nblk):
                cps.append(pltpu.async_copy(
                    bufs[b].at[j],
                    shared.at[pl.ds(j * V + col0 + b * 128, 128)], ssem))
            return cps

        @pl.when(sid < 15)
        def _():
            for cp in push_row(NBLK):
                cp.wait()

        @pl.when(sid == 15)
        def _():
            cps = push_row(NBLK_LAST)
            cps.append(pltpu.async_copy(
                tail_v.at[j],
                shared.at[pl.ds(j * V + col0 + NBLK_LAST * 128, TAIL)],
                ssem))
            for cp in cps:
                cp.wait()

        plsc.subcore_barrier()

        for c in range(BPW // 128):
            gathers.append(pltpu.async_copy(
                shared.at[cola_v.at[j, pl.ds(c * 128, 128)]],
                out_v.at[3 + j, pl.ds(c * 128, 128)], gsem))
            gathers.append(pltpu.async_copy(
                shared.at[seca_v.at[j, pl.ds(c * 128, 128)]],
                out_v.at[11 + j, pl.ds(c * 128, 128)], gsem))

    for cp in gathers:
        cp.wait()

    pltpu.sync_copy(out_v, out_h.at[:, pl.ds(base, BPW)])


def kernel(op_idx, col_idx, sec_idx, flag, op_emb, col_emb):
    col_t = col_emb.T
    out_t = _encode(
        op_idx.astype(jnp.int32),
        col_idx.astype(jnp.int32),
        sec_idx.astype(jnp.int32),
        flag.reshape(B).astype(jnp.float32),
        op_emb,
        col_t,
        col_t[:, 15 * CH + NBLK_LAST * 128:],
    )
    return out_t.T


# speedup vs baseline: 4.3728x; 1.1154x over previous
import functools

import jax
import jax.numpy as jnp
from jax import lax
from jax.experimental import pallas as pl
from jax.experimental.pallas import tpu as pltpu
from jax.experimental.pallas import tpu_sc as plsc

B = 16384
NC = 2
NS = 16
NW = NC * NS
BPW = B // NW
V = 100000
NBLK = 49
NBLK_LAST = 46
CH = NBLK * 128
TAIL = V - 15 * CH - NBLK_LAST * 128
OUT_D = 20

_mesh = plsc.VectorSubcoreMesh(
    core_axis_name="c", subcore_axis_name="s", num_cores=NC, num_subcores=NS
)


@functools.partial(
    pl.kernel,
    out_type=jax.ShapeDtypeStruct((OUT_D, B), jnp.float32),
    mesh=_mesh,
    compiler_params=pltpu.CompilerParams(
        needs_layout_passes=False, use_tc_tiling_on_sc=True,
        skip_device_barrier=True, disable_bounds_checks=True,
        disable_semaphore_checks=True),
    scratch_types=[
        pltpu.VMEM_SHARED((8 * V,), jnp.float32),
        [pltpu.VMEM((8, 128), jnp.float32)] * NBLK,
        pltpu.VMEM((8, TAIL), jnp.float32),
        pltpu.VMEM((BPW,), jnp.int32),
        pltpu.VMEM((BPW,), jnp.int32),
        pltpu.VMEM((BPW,), jnp.int32),
        pltpu.VMEM((8, BPW), jnp.int32),
        pltpu.VMEM((8, BPW), jnp.int32),
        pltpu.VMEM((6, 3), jnp.float32),
        pltpu.VMEM((OUT_D, BPW), jnp.float32),
        pltpu.SemaphoreType.DMA,
        pltpu.SemaphoreType.DMA,
        pltpu.SemaphoreType.DMA,
        pltpu.SemaphoreType.DMA,
    ],
)
def _encode(op_idx_h, col_idx_h, sec_idx_h, flag_h, op_emb_h, col_t_h,
            tail_h, out_h, shared, bufs, tail_v, opi_v, coli_v,
            seci_v, cola_v, seca_v, opt_v, out_v, gsem, hsem, ssem0, ssem1):
    cid = lax.axis_index("c")
    sid = lax.axis_index("s")
    wid = sid * NC + cid
    base = wid * BPW
    col0 = sid * CH

    def fetch(b):
        return pltpu.async_copy(
            col_t_h.at[:, pl.ds(col0 + b * 128, 128)], bufs[b], hsem)

    fetches = [fetch(b) for b in range(NBLK_LAST)]

    @pl.when(sid < 15)
    def _():
        for b in range(NBLK_LAST, NBLK):
            fetch(b)

    @pl.when(sid == 15)
    def _():
        pltpu.sync_copy(tail_h, tail_v)

    pltpu.sync_copy(col_idx_h.at[pl.ds(base, BPW)], coli_v)
    pltpu.sync_copy(sec_idx_h.at[pl.ds(base, BPW)], seci_v)
    pltpu.sync_copy(op_idx_h.at[pl.ds(base, BPW)], opi_v)
    pltpu.sync_copy(op_emb_h, opt_v)
    pltpu.sync_copy(flag_h.at[pl.ds(base, BPW)], out_v.at[19])

    def prep_step(i, carry):
        ci = coli_v[pl.ds(i * 16, 16)]
        si = seci_v[pl.ds(i * 16, 16)]
        for j in range(8):
            cola_v[j, pl.ds(i * 16, 16)] = ci + j * V
            seca_v[j, pl.ds(i * 16, 16)] = si + j * V
        o = opi_v[pl.ds(i * 16, 16)]
        for j in range(3):
            v = plsc.load_gather(opt_v, [o, jnp.full((16,), j, jnp.int32)])
            out_v[j, pl.ds(i * 16, 16)] = v
        return carry

    lax.fori_loop(0, BPW // 16, prep_step, 0)

    for cp in fetches:
        cp.wait()

    @pl.when(sid < 15)
    def _():
        for b in range(NBLK_LAST, NBLK):
            pltpu.make_async_copy(
                col_t_h.at[:, pl.ds(col0 + b * 128, 128)], bufs[b],
                hsem).wait()

    def push_row(j):
        sem = ssem0 if j % 2 == 0 else ssem1
        cps = []
        for b in range(NBLK_LAST):
            cps.append(pltpu.async_copy(
                bufs[b].at[j],
                shared.at[pl.ds(j * V + col0 + b * 128, 128)], sem))

        @pl.when(sid < 15)
        def _():
            for b in range(NBLK_LAST, NBLK):
                pltpu.async_copy(
                    bufs[b].at[j],
                    shared.at[pl.ds(j * V + col0 + b * 128, 128)], sem)

        @pl.when(sid == 15)
        def _():
            pltpu.async_copy(
                tail_v.at[j],
                shared.at[pl.ds(j * V + col0 + NBLK_LAST * 128, TAIL)], sem)
        return cps

    def drain_row(j, cps):
        sem = ssem0 if j % 2 == 0 else ssem1
        for cp in cps:
            cp.wait()

        @pl.when(sid < 15)
        def _():
            for b in range(NBLK_LAST, NBLK):
                pltpu.make_async_copy(
                    bufs[b].at[j],
                    shared.at[pl.ds(j * V + col0 + b * 128, 128)],
                    sem).wait()

        @pl.when(sid == 15)
        def _():
            pltpu.make_async_copy(
                tail_v.at[j],
                shared.at[pl.ds(j * V + col0 + NBLK_LAST * 128, TAIL)],
                sem).wait()

    gathers = []
    pend = push_row(0)
    for j in range(8):
        nxt = push_row(j + 1) if j + 1 < 8 else None
        drain_row(j, pend)
        pend = nxt

        plsc.subcore_barrier()

        for c in range(BPW // 128):
            gathers.append(pltpu.async_copy(
                shared.at[cola_v.at[j, pl.ds(c * 128, 128)]],
                out_v.at[3 + j, pl.ds(c * 128, 128)], gsem))
            gathers.append(pltpu.async_copy(
                shared.at[seca_v.at[j, pl.ds(c * 128, 128)]],
                out_v.at[11 + j, pl.ds(c * 128, 128)], gsem))

    for cp in gathers:
        cp.wait()

    pltpu.sync_copy(out_v, out_h.at[:, pl.ds(base, BPW)])


def kernel(op_idx, col_idx, sec_idx, flag, op_emb, col_emb):
    col_t = col_emb.T
    out_t = _encode(
        op_idx.astype(jnp.int32),
        col_idx.astype(jnp.int32),
        sec_idx.astype(jnp.int32),
        flag.reshape(B).astype(jnp.float32),
        op_emb,
        col_t,
        col_t[:, 15 * CH + NBLK_LAST * 128:],
    )
    return out_t.T


# speedup vs baseline: 4.4147x; 1.0096x over previous
import functools

import jax
import jax.numpy as jnp
from jax import lax
from jax.experimental import pallas as pl
from jax.experimental.pallas import tpu as pltpu
from jax.experimental.pallas import tpu_sc as plsc

B = 16384
NC = 2
NS = 16
NW = NC * NS
BPW = B // NW
V = 100000
VP = 100096
NBLK = 49
NBLK_LAST = 46
CH = NBLK * 128
TAILOFF = 15 * CH + NBLK_LAST * 128
OUT_D = 20

_mesh = plsc.VectorSubcoreMesh(
    core_axis_name="c", subcore_axis_name="s", num_cores=NC, num_subcores=NS
)


@functools.partial(
    pl.kernel,
    out_type=jax.ShapeDtypeStruct((OUT_D, B), jnp.float32),
    mesh=_mesh,
    compiler_params=pltpu.CompilerParams(
        needs_layout_passes=False, use_tc_tiling_on_sc=True,
        skip_device_barrier=True, disable_bounds_checks=True,
        disable_semaphore_checks=True),
    scratch_types=[
        pltpu.VMEM_SHARED((8 * VP,), jnp.float32),
        [pltpu.VMEM((8, 128), jnp.float32)] * NBLK,
        pltpu.VMEM((1042,), jnp.float32),
        pltpu.VMEM((BPW,), jnp.int32),
        pltpu.VMEM((BPW,), jnp.int32),
        pltpu.VMEM((BPW,), jnp.int32),
        pltpu.VMEM((8, BPW), jnp.int32),
        pltpu.VMEM((8, BPW), jnp.int32),
        pltpu.VMEM((OUT_D, BPW), jnp.float32),
        pltpu.SemaphoreType.DMA,
        pltpu.SemaphoreType.DMA,
        pltpu.SemaphoreType.DMA,
        pltpu.SemaphoreType.DMA,
    ],
)
def _encode(op_idx_h, col_idx_h, sec_idx_h, flag_h, aux_h, col_t_h,
            out_h, shared, bufs, aux_v, opi_v, coli_v,
            seci_v, cola_v, seca_v, out_v, gsem, hsem, ssem0, ssem1):
    cid = lax.axis_index("c")
    sid = lax.axis_index("s")
    wid = sid * NC + cid
    base = wid * BPW
    col0 = sid * CH

    def fetch(b):
        return pltpu.async_copy(
            col_t_h.at[:, pl.ds(col0 + b * 128, 128)], bufs[b], hsem)

    fetches = [fetch(b) for b in range(NBLK_LAST)]

    @pl.when(sid < 15)
    def _():
        for b in range(NBLK_LAST, NBLK):
            fetch(b)

    pltpu.sync_copy(col_idx_h.at[pl.ds(base, BPW)], coli_v)
    pltpu.sync_copy(sec_idx_h.at[pl.ds(base, BPW)], seci_v)
    pltpu.sync_copy(op_idx_h.at[pl.ds(base, BPW)], opi_v)
    pltpu.sync_copy(aux_h, aux_v)
    pltpu.sync_copy(flag_h.at[pl.ds(base, BPW)], out_v.at[19])

    def prep_step(i, carry):
        ci = coli_v[pl.ds(i * 16, 16)]
        si = seci_v[pl.ds(i * 16, 16)]
        for j in range(8):
            cola_v[j, pl.ds(i * 16, 16)] = ci + j * VP
            seca_v[j, pl.ds(i * 16, 16)] = si + j * VP
        o = opi_v[pl.ds(i * 16, 16)]
        for j in range(3):
            v = plsc.load_gather(aux_v, [o + (1024 + 6 * j)])
            out_v[j, pl.ds(i * 16, 16)] = v
        return carry

    lax.fori_loop(0, BPW // 16, prep_step, 0)

    for cp in fetches:
        cp.wait()

    @pl.when(sid < 15)
    def _():
        for b in range(NBLK_LAST, NBLK):
            pltpu.make_async_copy(
                col_t_h.at[:, pl.ds(col0 + b * 128, 128)], bufs[b],
                hsem).wait()

    def push_row(j):
        sem = ssem0 if j % 2 == 0 else ssem1
        cps = []
        for b in range(NBLK_LAST):
            cps.append(pltpu.async_copy(
                bufs[b].at[j],
                shared.at[pl.ds(j * VP + col0 + b * 128, 128)], sem))

        @pl.when(sid < 15)
        def _():
            for b in range(NBLK_LAST, NBLK):
                pltpu.async_copy(
                    bufs[b].at[j],
                    shared.at[pl.ds(j * VP + col0 + b * 128, 128)], sem)

        @pl.when(sid == 15)
        def _():
            pltpu.async_copy(
                aux_v.at[pl.ds(j * 128, 128)],
                shared.at[pl.ds(j * VP + TAILOFF, 128)], sem)
        return cps

    def drain_row(j, cps):
        sem = ssem0 if j % 2 == 0 else ssem1
        for cp in cps:
            cp.wait()

        @pl.when(sid < 15)
        def _():
            for b in range(NBLK_LAST, NBLK):
                pltpu.make_async_copy(
                    bufs[b].at[j],
                    shared.at[pl.ds(j * VP + col0 + b * 128, 128)],
                    sem).wait()

        @pl.when(sid == 15)
        def _():
            pltpu.make_async_copy(
                aux_v.at[pl.ds(j * 128, 128)],
                shared.at[pl.ds(j * VP + TAILOFF, 128)],
                sem).wait()

    gathers = []
    pend = push_row(0)
    for j in range(8):
        nxt = push_row(j + 1) if j + 1 < 8 else None
        drain_row(j, pend)
        pend = nxt

        plsc.subcore_barrier()

        for c in range(BPW // 128):
            gathers.append(pltpu.async_copy(
                shared.at[cola_v.at[j, pl.ds(c * 128, 128)]],
                out_v.at[3 + j, pl.ds(c * 128, 128)], gsem))
            gathers.append(pltpu.async_copy(
                shared.at[seca_v.at[j, pl.ds(c * 128, 128)]],
                out_v.at[11 + j, pl.ds(c * 128, 128)], gsem))

    for cp in gathers:
        cp.wait()

    pltpu.sync_copy(out_v, out_h.at[:, pl.ds(base, BPW)])


def kernel(op_idx, col_idx, sec_idx, flag, op_emb, col_emb):
    col_t = col_emb.T
    tail_pad = jnp.pad(col_t[:, TAILOFF:], ((0, 0), (0, 128 - (V - TAILOFF))))
    aux = jnp.concatenate([tail_pad.reshape(1024), op_emb.T.reshape(18)])
    out_t = _encode(
        op_idx.astype(jnp.int32),
        col_idx.astype(jnp.int32),
        sec_idx.astype(jnp.int32),
        flag.reshape(B).astype(jnp.float32),
        aux,
        col_t,
    )
    return out_t.T
